# baseline (speedup 1.0000x reference)
import functools

import jax
import jax.numpy as jnp
from jax import lax
from jax.experimental import pallas as pl
from jax.experimental.pallas import tpu as pltpu
from jax.experimental.pallas import tpu_sc as plsc

D = 64
ROWS = 4096
COLS = 200
B = ROWS * COLS
NC = 2
NS = 16
NW = NC * NS
BPW = B // NW
GROUP = 1024
SEG = 128
K = GROUP // SEG
NG = BPW // GROUP
SCALE = 8.0


def _gather_scaled(x_flat, lut):
    mesh = plsc.VectorSubcoreMesh(core_axis_name="c", subcore_axis_name="s")

    @functools.partial(
        pl.kernel,
        mesh=mesh,
        out_type=jax.ShapeDtypeStruct((B, D), jnp.float32),
        scratch_types=[
            pltpu.VMEM((GROUP,), jnp.int32),
            pltpu.VMEM((GROUP, D), jnp.float32),
            pltpu.SemaphoreType.DMA,
        ],
        compiler_params=pltpu.CompilerParams(use_tc_tiling_on_sc=False),
    )
    def k(idx_hbm, table_hbm, out_hbm, idx_v, rows_v, sem):
        wid = lax.axis_index("s") * NC + lax.axis_index("c")
        base = wid * BPW

        def step(g, carry):
            off = base + g * GROUP
            pltpu.sync_copy(idx_hbm.at[pl.ds(off, GROUP)], idx_v)
            copies = []
            for j in range(K):
                copies.append(pltpu.async_copy(
                    table_hbm.at[idx_v.at[pl.ds(j * SEG, SEG)]],
                    rows_v.at[pl.ds(j * SEG, SEG)],
                    sem,
                ))
            for c in copies:
                c.wait()

            def scale_row(r, c2):
                for q in range(D // 16):
                    sl = pl.ds(q * 16, 16)
                    rows_v[r, sl] = rows_v[r, sl] * SCALE
                return c2

            lax.fori_loop(0, GROUP, scale_row, 0)
            pltpu.sync_copy(rows_v, out_hbm.at[pl.ds(off, GROUP)])
            return carry

        lax.fori_loop(0, NG, step, 0)

    return k(x_flat, lut)


def kernel(x, lut):
    x_flat = x.reshape(B).astype(jnp.int32)
    out = _gather_scaled(x_flat, lut)
    return out.reshape(ROWS, COLS, D)


# speedup vs baseline: 1.0445x; 1.0445x over previous
import functools

import jax
import jax.numpy as jnp
from jax import lax
from jax.experimental import pallas as pl
from jax.experimental.pallas import tpu as pltpu
from jax.experimental.pallas import tpu_sc as plsc

D = 64
ROWS = 4096
COLS = 200
B = ROWS * COLS
NC = 2
NS = 16
NW = NC * NS
BPW = B // NW
GROUP = 512
SEG = 128
K = GROUP // SEG
NG = BPW // GROUP
NPAIR = NG // 2
SCALE = 8.0


def _gather_scaled(x_flat, lut):
    mesh = plsc.VectorSubcoreMesh(core_axis_name="c", subcore_axis_name="s")

    @functools.partial(
        pl.kernel,
        mesh=mesh,
        out_type=jax.ShapeDtypeStruct((B, D), jnp.float32),
        scratch_types=[
            pltpu.VMEM((2, GROUP), jnp.int32),
            pltpu.VMEM((2, GROUP, D), jnp.float32),
            pltpu.SemaphoreType.DMA,
            pltpu.SemaphoreType.DMA,
            pltpu.SemaphoreType.DMA,
            pltpu.SemaphoreType.DMA,
        ],
        compiler_params=pltpu.CompilerParams(use_tc_tiling_on_sc=False),
    )
    def k(idx_hbm, table_hbm, out_hbm, idx_v, rows_v, g0, g1, s0, s1):
        wid = lax.axis_index("s") * NC + lax.axis_index("c")
        base = wid * BPW
        gsem = (g0, g1)
        ssem = (s0, s1)

        def fire_gather(g, b):
            off = base + g * GROUP
            pltpu.sync_copy(idx_hbm.at[pl.ds(off, GROUP)], idx_v.at[b])
            for j in range(K):
                pltpu.async_copy(
                    table_hbm.at[idx_v.at[b, pl.ds(j * SEG, SEG)]],
                    rows_v.at[b, pl.ds(j * SEG, SEG)],
                    gsem[b],
                )

        def wait_gather(b):
            pltpu.make_async_copy(
                out_hbm.at[pl.ds(0, GROUP)], rows_v.at[b], gsem[b]
            ).wait()

        def start_store(g, b):
            off = base + g * GROUP
            pltpu.async_copy(rows_v.at[b], out_hbm.at[pl.ds(off, GROUP)], ssem[b])

        def wait_store(b):
            pltpu.make_async_copy(
                rows_v.at[b], out_hbm.at[pl.ds(0, GROUP)], ssem[b]
            ).wait()

        def scale(b):
            @plsc.parallel_loop(0, GROUP, step=1, unroll=8)
            def _(r):
                for q in range(D // 16):
                    sl = pl.ds(q * 16, 16)
                    rows_v[b, r, sl] = rows_v[b, r, sl] * SCALE

        fire_gather(0, 0)

        def step(t, carry):
            ge = 2 * t
            wait_gather(0)
            scale(0)

            @pl.when(t > 0)
            def _():
                wait_store(1)

            fire_gather(ge + 1, 1)
            start_store(ge, 0)
            wait_gather(1)
            scale(1)
            wait_store(0)

            @pl.when(t < NPAIR - 1)
            def _():
                fire_gather(ge + 2, 0)

            start_store(ge + 1, 1)
            return carry

        lax.fori_loop(0, NPAIR, step, 0)
        wait_store(1)

    return k(x_flat, lut)


def kernel(x, lut):
    x_flat = x.reshape(B).astype(jnp.int32)
    out = _gather_scaled(x_flat, lut)
    return out.reshape(ROWS, COLS, D)


# speedup vs baseline: 1.0597x; 1.0146x over previous
import functools

import jax
import jax.numpy as jnp
from jax import lax
from jax.experimental import pallas as pl
from jax.experimental.pallas import tpu as pltpu
from jax.experimental.pallas import tpu_sc as plsc

D = 64
ROWS = 4096
COLS = 200
NC = 2
NS = 16
NW = NC * NS
RPW = ROWS // NW
GR = 4
GROUP = GR * COLS
NG = RPW // GR
NPAIR = NG // 2
SCALE = 8.0


def _gather_scaled(x, lut):
    mesh = plsc.VectorSubcoreMesh(core_axis_name="c", subcore_axis_name="s")

    @functools.partial(
        pl.kernel,
        mesh=mesh,
        out_type=jax.ShapeDtypeStruct((ROWS, COLS, D), jnp.float32),
        scratch_types=[
            pltpu.VMEM((2, GR, COLS), jnp.int32),
            pltpu.VMEM((2, GR, COLS, D), jnp.float32),
            pltpu.SemaphoreType.DMA,
            pltpu.SemaphoreType.DMA,
            pltpu.SemaphoreType.DMA,
            pltpu.SemaphoreType.DMA,
        ],
        compiler_params=pltpu.CompilerParams(use_tc_tiling_on_sc=False),
    )
    def k(idx_hbm, table_hbm, out_hbm, idx_v, rows_v, g0, g1, s0, s1):
        wid = lax.axis_index("s") * NC + lax.axis_index("c")
        base = wid * RPW
        gsem = (g0, g1)
        ssem = (s0, s1)

        def fire_gather(g, b):
            r0 = base + g * GR
            pltpu.sync_copy(idx_hbm.at[pl.ds(r0, GR), :], idx_v.at[b])
            for i in range(GR):
                pltpu.async_copy(
                    table_hbm.at[idx_v.at[b, i, pl.ds(0, 128)]],
                    rows_v.at[b, i, pl.ds(0, 128), :],
                    gsem[b],
                )
                pltpu.async_copy(
                    table_hbm.at[idx_v.at[b, i, pl.ds(128, COLS - 128)]],
                    rows_v.at[b, i, pl.ds(128, COLS - 128), :],
                    gsem[b],
                )

        def wait_gather(b):
            pltpu.make_async_copy(
                out_hbm.at[pl.ds(0, GR)], rows_v.at[b], gsem[b]
            ).wait()

        def start_store(g, b):
            r0 = base + g * GR
            pltpu.async_copy(rows_v.at[b], out_hbm.at[pl.ds(r0, GR)], ssem[b])

        def wait_store(b):
            pltpu.make_async_copy(
                rows_v.at[b], out_hbm.at[pl.ds(0, GR)], ssem[b]
            ).wait()

        def scale(b):
            for i in range(GR):
                @plsc.parallel_loop(0, COLS, step=1, unroll=8)
                def _(r):
                    for q in range(D // 16):
                        sl = pl.ds(q * 16, 16)
                        rows_v[b, i, r, sl] = rows_v[b, i, r, sl] * SCALE

        fire_gather(0, 0)

        def step(t, carry):
            ge = 2 * t
            wait_gather(0)
            scale(0)

            @pl.when(t > 0)
            def _():
                wait_store(1)

            fire_gather(ge + 1, 1)
            start_store(ge, 0)
            wait_gather(1)
            scale(1)
            wait_store(0)

            @pl.when(t < NPAIR - 1)
            def _():
                fire_gather(ge + 2, 0)

            start_store(ge + 1, 1)
            return carry

        lax.fori_loop(0, NPAIR, step, 0)
        wait_store(1)

    return k(x, lut)


def kernel(x, lut):
    return _gather_scaled(x, lut)
